# Initial kernel scaffold; baseline (speedup 1.0000x reference)
import jax
import jax.numpy as jnp
from jax.experimental import pallas as pl


def kernel(images):
    raise NotImplementedError("write your pallas kernel here")


# speedup vs baseline: 6.6451x; 6.6451x over previous
import functools

import jax
import jax.numpy as jnp
from jax.experimental import pallas as pl
from jax.experimental.pallas import tpu as pltpu

_BIG = 1e30


def _softplus(x):
    return jnp.maximum(x, 0.0) + jnp.log1p(jnp.exp(-jnp.abs(x)))


def _shift_right(x, d, fill):
    b = x.shape[0]
    pad = jnp.full((b, d), fill, dtype=x.dtype)
    return jnp.concatenate([pad, x[:, :-d]], axis=1)


def _cumsum(x):
    n = x.shape[-1]
    d = 1
    while d < n:
        x = x + _shift_right(x, d, 0.0)
        d *= 2
    return x


def _cummin(x):
    n = x.shape[-1]
    d = 1
    while d < n:
        x = jnp.minimum(x, _shift_right(x, d, _BIG))
        d *= 2
    return x


def _dp_body(img_ref, out_ref, prev_img, carry):
    i = pl.program_id(0)
    cur = img_ref[0]

    @pl.when(i == 0)
    def _init():
        left = cur
        right = jnp.concatenate([cur[:, 1:], cur[:, -1:]], axis=1)
        wr = _softplus((left + right) * 0.5)
        carry[...] = _shift_right(_cumsum(wr), 1, 0.0)
        prev_img[...] = cur

    @pl.when(i > 0)
    def _step():
        prev = prev_img[...]
        prev_d = carry[...]
        wd = _softplus((prev + cur) * 0.5)
        cand_down = prev_d + wd
        prev_s = _shift_right(prev, 1, _BIG)
        prev_d_s = _shift_right(prev_d, 1, _BIG)
        wdg = _softplus((prev_s + cur) * 0.5)
        cand_diag = prev_d_s + wdg
        a = jnp.minimum(cand_down, cand_diag)
        right = jnp.concatenate([cur[:, 1:], cur[:, -1:]], axis=1)
        wr = _softplus((cur + right) * 0.5)
        p = _shift_right(_cumsum(wr), 1, 0.0)
        d = p + _cummin(a - p)
        carry[...] = d
        prev_img[...] = cur

    @pl.when(i == pl.num_programs(0) - 1)
    def _emit():
        out_ref[...] = carry[...]


@jax.jit
def kernel(images):
    b, h, w = images.shape
    imgs_t = images.transpose(1, 0, 2)
    out = pl.pallas_call(
        _dp_body,
        grid=(h,),
        in_specs=[pl.BlockSpec((1, b, w), lambda i: (i, 0, 0))],
        out_specs=pl.BlockSpec((b, w), lambda i: (0, 0)),
        out_shape=jax.ShapeDtypeStruct((b, w), jnp.float32),
        scratch_shapes=[
            pltpu.VMEM((b, w), jnp.float32),
            pltpu.VMEM((b, w), jnp.float32),
        ],
    )(imgs_t)
    return out[:, -1]


# speedup vs baseline: 11.5566x; 1.7391x over previous
import functools

import jax
import jax.numpy as jnp
from jax.experimental import pallas as pl
from jax.experimental.pallas import tpu as pltpu

_BIG = 1e30
_ROWS = 8


def _softplus(x):
    return jnp.maximum(x, 0.0) + jnp.log1p(jnp.exp(-jnp.abs(x)))


def _shift_right(x, d, fill):
    b = x.shape[0]
    pad = jnp.full((b, d), fill, dtype=x.dtype)
    return jnp.concatenate([pad, x[:, :-d]], axis=1)


def _cumsum(x):
    n = x.shape[-1]
    d = 1
    while d < n:
        x = x + _shift_right(x, d, 0.0)
        d *= 2
    return x


def _cummin(x):
    n = x.shape[-1]
    d = 1
    while d < n:
        x = jnp.minimum(x, _shift_right(x, d, _BIG))
        d *= 2
    return x


def _first_row(cur):
    right = jnp.concatenate([cur[:, 1:], cur[:, -1:]], axis=1)
    wr = _softplus((cur + right) * 0.5)
    return _shift_right(_cumsum(wr), 1, 0.0)


def _row_update(prev_im, cur_im, prev_d):
    wd = _softplus((prev_im + cur_im) * 0.5)
    cand_down = prev_d + wd
    prev_im_s = _shift_right(prev_im, 1, _BIG)
    prev_d_s = _shift_right(prev_d, 1, _BIG)
    wdg = _softplus((prev_im_s + cur_im) * 0.5)
    a = jnp.minimum(cand_down, prev_d_s + wdg)
    right = jnp.concatenate([cur_im[:, 1:], cur_im[:, -1:]], axis=1)
    wr = _softplus((cur_im + right) * 0.5)
    p = _shift_right(_cumsum(wr), 1, 0.0)
    return p + _cummin(a - p)


def _dp_body(img_ref, out_ref, prev_img, carry):
    g = pl.program_id(0)
    cur = img_ref[...]
    rows = [cur[r] for r in range(_ROWS)]

    @pl.when(g == 0)
    def _init():
        d = _first_row(rows[0])
        for r in range(1, _ROWS):
            d = _row_update(rows[r - 1], rows[r], d)
        carry[...] = d
        prev_img[...] = rows[_ROWS - 1]

    @pl.when(g > 0)
    def _step():
        d = carry[...]
        pim = prev_img[...]
        for r in range(_ROWS):
            d = _row_update(pim, rows[r], d)
            pim = rows[r]
        carry[...] = d
        prev_img[...] = pim

    @pl.when(g == pl.num_programs(0) - 1)
    def _emit():
        out_ref[...] = carry[...]


@jax.jit
def kernel(images):
    b, h, w = images.shape
    imgs_t = images.transpose(1, 0, 2)
    out = pl.pallas_call(
        _dp_body,
        grid=(h // _ROWS,),
        in_specs=[pl.BlockSpec((_ROWS, b, w), lambda g: (g, 0, 0))],
        out_specs=pl.BlockSpec((b, w), lambda g: (0, 0)),
        out_shape=jax.ShapeDtypeStruct((b, w), jnp.float32),
        scratch_shapes=[
            pltpu.VMEM((b, w), jnp.float32),
            pltpu.VMEM((b, w), jnp.float32),
        ],
    )(imgs_t)
    return out[:, -1]


# speedup vs baseline: 11.9522x; 1.0342x over previous
import functools

import jax
import jax.numpy as jnp
from jax.experimental import pallas as pl
from jax.experimental.pallas import tpu as pltpu

_BIG = 1e30
_ROWS = 8


def _softplus(x):
    return jnp.maximum(x, 0.0) + jnp.log1p(jnp.exp(-jnp.abs(x)))


def _shift_right(x, d, fill):
    b = x.shape[0]
    pad = jnp.full((b, d), fill, dtype=x.dtype)
    return jnp.concatenate([pad, x[:, :-d]], axis=1)


def _cumsum(x):
    n = x.shape[-1]
    d = 1
    while d < n:
        parts = [_shift_right(x, d * k, 0.0) for k in range(1, 8) if d * k < n]
        for p_ in parts:
            x = x + p_
        d *= 8
    return x


def _cummin(x):
    n = x.shape[-1]
    d = 1
    while d < n:
        parts = [x] + [_shift_right(x, d * k, _BIG) for k in range(1, 8) if d * k < n]
        while len(parts) > 1:
            parts = [jnp.minimum(parts[i], parts[i + 1]) if i + 1 < len(parts)
                     else parts[i] for i in range(0, len(parts), 2)]
        x = parts[0]
        d *= 8
    return x


def _first_row(cur):
    right = jnp.concatenate([cur[:, 1:], cur[:, -1:]], axis=1)
    wr = _softplus((cur + right) * 0.5)
    return _shift_right(_cumsum(wr), 1, 0.0)


def _row_update(prev_im, cur_im, prev_d):
    wd = _softplus((prev_im + cur_im) * 0.5)
    cand_down = prev_d + wd
    prev_im_s = _shift_right(prev_im, 1, _BIG)
    prev_d_s = _shift_right(prev_d, 1, _BIG)
    wdg = _softplus((prev_im_s + cur_im) * 0.5)
    a = jnp.minimum(cand_down, prev_d_s + wdg)
    right = jnp.concatenate([cur_im[:, 1:], cur_im[:, -1:]], axis=1)
    wr = _softplus((cur_im + right) * 0.5)
    p = _shift_right(_cumsum(wr), 1, 0.0)
    return p + _cummin(a - p)


def _dp_body(img_ref, out_ref, prev_img, carry):
    g = pl.program_id(0)
    cur = img_ref[...]
    rows = [cur[r] for r in range(_ROWS)]

    @pl.when(g == 0)
    def _init():
        d = _first_row(rows[0])
        for r in range(1, _ROWS):
            d = _row_update(rows[r - 1], rows[r], d)
        carry[...] = d
        prev_img[...] = rows[_ROWS - 1]

    @pl.when(g > 0)
    def _step():
        d = carry[...]
        pim = prev_img[...]
        for r in range(_ROWS):
            d = _row_update(pim, rows[r], d)
            pim = rows[r]
        carry[...] = d
        prev_img[...] = pim

    @pl.when(g == pl.num_programs(0) - 1)
    def _emit():
        out_ref[...] = carry[...]


@jax.jit
def kernel(images):
    b, h, w = images.shape
    imgs_t = images.transpose(1, 0, 2)
    out = pl.pallas_call(
        _dp_body,
        grid=(h // _ROWS,),
        in_specs=[pl.BlockSpec((_ROWS, b, w), lambda g: (g, 0, 0))],
        out_specs=pl.BlockSpec((b, w), lambda g: (0, 0)),
        out_shape=jax.ShapeDtypeStruct((b, w), jnp.float32),
        scratch_shapes=[
            pltpu.VMEM((b, w), jnp.float32),
            pltpu.VMEM((b, w), jnp.float32),
        ],
    )(imgs_t)
    return out[:, -1]


# speedup vs baseline: 14.1577x; 1.1845x over previous
import functools

import jax
import jax.numpy as jnp
from jax import lax
from jax.experimental import pallas as pl
from jax.experimental.pallas import tpu as pltpu

_BIG = 1e30
_ROWS = 8


def _softplus(x):
    return jnp.maximum(x, 0.0) + jnp.log1p(jnp.exp(-jnp.abs(x)))


def _shift_right(x, d, fill):
    b = x.shape[0]
    pad = jnp.full((b, d), fill, dtype=x.dtype)
    return jnp.concatenate([pad, x[:, :-d]], axis=1)


def _cummin(x, lo=0):
    n = x.shape[-1]
    parts = ([x] if lo == 0 else []) + [
        _shift_right(x, k, _BIG) for k in range(max(lo, 1), 9 - lo)
    ]
    x = functools.reduce(jnp.minimum, parts)
    d = 8
    while d < n:
        parts = [x] + [_shift_right(x, d * k, _BIG) for k in range(1, 8) if d * k < n]
        x = functools.reduce(jnp.minimum, parts)
        d *= 8
    return x


def _excl_prefix_sum(wr, tri):
    return lax.dot_general(
        wr, tri, (((1,), (0,)), ((), ())),
        precision=lax.Precision.HIGHEST,
        preferred_element_type=jnp.float32,
    )


def _row_update(prev_im, cur_im, prev_d, tri):
    cur_l = jnp.concatenate([cur_im[:, 1:], cur_im[:, -1:]], axis=1)
    wd = _softplus((prev_im + cur_im) * 0.5)
    wdgl = _softplus((prev_im + cur_l) * 0.5)
    wr = _softplus((cur_im + cur_l) * 0.5)
    p = _excl_prefix_sum(wr, tri)
    m1 = _cummin(prev_d + (wd - p))
    m2 = _cummin(prev_d + (wdgl - (p + wr)), lo=1)
    return p + jnp.minimum(m1, m2)


def _first_row(cur, tri):
    right = jnp.concatenate([cur[:, 1:], cur[:, -1:]], axis=1)
    wr = _softplus((cur + right) * 0.5)
    return _excl_prefix_sum(wr, tri)


def _dp_body(tri_ref, img_ref, out_ref, prev_img, carry):
    g = pl.program_id(0)
    cur = img_ref[...]
    tri = tri_ref[...]
    rows = [cur[r] for r in range(_ROWS)]

    @pl.when(g == 0)
    def _init():
        d = _first_row(rows[0], tri)
        for r in range(1, _ROWS):
            d = _row_update(rows[r - 1], rows[r], d, tri)
        carry[...] = d
        prev_img[...] = rows[_ROWS - 1]

    @pl.when(g > 0)
    def _step():
        d = carry[...]
        pim = prev_img[...]
        for r in range(_ROWS):
            d = _row_update(pim, rows[r], d, tri)
            pim = rows[r]
        carry[...] = d
        prev_img[...] = pim

    @pl.when(g == pl.num_programs(0) - 1)
    def _emit():
        out_ref[...] = carry[...]


@jax.jit
def kernel(images):
    b, h, w = images.shape
    imgs_t = images.transpose(1, 0, 2)
    tri = jnp.triu(jnp.ones((w, w), jnp.float32), k=1)
    out = pl.pallas_call(
        _dp_body,
        grid=(h // _ROWS,),
        in_specs=[
            pl.BlockSpec((w, w), lambda g: (0, 0)),
            pl.BlockSpec((_ROWS, b, w), lambda g: (g, 0, 0)),
        ],
        out_specs=pl.BlockSpec((b, w), lambda g: (0, 0)),
        out_shape=jax.ShapeDtypeStruct((b, w), jnp.float32),
        scratch_shapes=[
            pltpu.VMEM((b, w), jnp.float32),
            pltpu.VMEM((b, w), jnp.float32),
        ],
    )(tri, imgs_t)
    return out[:, -1]
